# baseline (speedup 1.0000x reference)
import functools

import jax
import jax.numpy as jnp
from jax import lax
from jax.experimental import pallas as pl
from jax.experimental.pallas import tpu as pltpu
from jax.experimental.pallas import tpu_sc as plsc

VOCAB = 100000
EMBED_DIM = 300
DP = 304
BATCH = 4096
SEQ = 200

NUM_CORES = 2
NUM_SUBCORES = 16
NW = NUM_CORES * NUM_SUBCORES

TOTAL = BATCH * SEQ
B_PER_W = TOTAL // NW
CHUNK = 128
N_CHUNKS = B_PER_W // CHUNK
DENSE = CHUNK * EMBED_DIM
N_BLK = (EMBED_DIM + 15) // 16


def _make_sc_gather():
  mesh = plsc.VectorSubcoreMesh(
      core_axis_name="c", subcore_axis_name="s",
      num_cores=NUM_CORES, num_subcores=NUM_SUBCORES)

  @functools.partial(
      pl.kernel,
      mesh=mesh,
      out_type=jax.ShapeDtypeStruct((TOTAL * EMBED_DIM,), jnp.float32),
      scratch_types=[
          pltpu.VMEM((CHUNK,), jnp.int32),
          pltpu.VMEM((CHUNK, DP), jnp.float32),
          pltpu.VMEM((DENSE + 16,), jnp.float32),
          pltpu.SemaphoreType.DMA,
      ],
      compiler_params=pltpu.CompilerParams(use_tc_tiling_on_sc=False),
  )
  def gather_kernel(tokens_hbm, table_hbm, out_hbm, idx_v, rows_v, dense_v,
                    sem):
    wid = lax.axis_index("s") * NUM_CORES + lax.axis_index("c")
    base = wid * B_PER_W

    def body(c, carry):
      pltpu.sync_copy(tokens_hbm.at[wid, c], idx_v)
      pltpu.async_copy(table_hbm.at[idx_v], rows_v, sem).wait()

      def row_body(r, carry2):
        db = r * EMBED_DIM
        for k in range(N_BLK):
          dense_v[pl.ds(db + 16 * k, 16)] = rows_v[r, pl.ds(16 * k, 16)]
        return carry2

      lax.fori_loop(0, CHUNK, row_body, 0)
      pltpu.sync_copy(
          dense_v.at[pl.ds(0, DENSE)],
          out_hbm.at[pl.ds((base + c * CHUNK) * EMBED_DIM, DENSE)])
      return carry

    lax.fori_loop(0, N_CHUNKS, body, 0)

  return gather_kernel


_sc_gather = _make_sc_gather()


def kernel(tokens, table):
  table_p = jnp.pad(table, ((0, 0), (0, DP - EMBED_DIM)))
  tokens_r = tokens.reshape(NW, N_CHUNKS, CHUNK)
  out = _sc_gather(tokens_r, table_p)
  return out.reshape(BATCH, SEQ, EMBED_DIM)


# speedup vs baseline: 1.1494x; 1.1494x over previous
import functools

import jax
import jax.numpy as jnp
from jax import lax
from jax.experimental import pallas as pl
from jax.experimental.pallas import tpu as pltpu
from jax.experimental.pallas import tpu_sc as plsc

VOCAB = 100000
EMBED_DIM = 300
DP = 304
BATCH = 4096
SEQ = 200

NUM_CORES = 2
NUM_SUBCORES = 16
NW = NUM_CORES * NUM_SUBCORES

TOTAL = BATCH * SEQ
B_PER_W = TOTAL // NW
CHUNK = 100
N_CHUNKS = B_PER_W // CHUNK
DENSE = CHUNK * EMBED_DIM
N_BLK = (EMBED_DIM + 15) // 16


def _make_sc_gather():
  mesh = plsc.VectorSubcoreMesh(
      core_axis_name="c", subcore_axis_name="s",
      num_cores=NUM_CORES, num_subcores=NUM_SUBCORES)

  @functools.partial(
      pl.kernel,
      mesh=mesh,
      out_type=jax.ShapeDtypeStruct((TOTAL * EMBED_DIM,), jnp.float32),
      scratch_types=[
          pltpu.VMEM((2, CHUNK), jnp.int32),
          pltpu.VMEM((CHUNK, DP), jnp.float32),
          pltpu.VMEM((CHUNK, DP), jnp.float32),
          pltpu.VMEM((DENSE + 16,), jnp.float32),
          pltpu.VMEM((DENSE + 16,), jnp.float32),
          pltpu.SemaphoreType.DMA,
          pltpu.SemaphoreType.DMA,
          pltpu.SemaphoreType.DMA,
          pltpu.SemaphoreType.DMA,
      ],
      compiler_params=pltpu.CompilerParams(use_tc_tiling_on_sc=False),
  )
  def gather_kernel(tokens_hbm, table_hbm, out_hbm, idx_v, rows0, rows1,
                    dense0, dense1, sg0, sg1, ss0, ss1):
    wid = lax.axis_index("s") * NUM_CORES + lax.axis_index("c")
    base = wid * B_PER_W
    rows = (rows0, rows1)
    dense = (dense0, dense1)
    sg = (sg0, sg1)
    ss = (ss0, ss1)

    def out_slice(c):
      return out_hbm.at[pl.ds((base + c * CHUNK) * EMBED_DIM, DENSE)]

    def repack(rows_v, dense_v):
      def row_body(r, carry2):
        db = r * EMBED_DIM
        for k in range(N_BLK):
          dense_v[pl.ds(db + 16 * k, 16)] = rows_v[r, pl.ds(16 * k, 16)]
        return carry2
      lax.fori_loop(0, CHUNK, row_body, 0)

    for b in (0, 1):
      pltpu.sync_copy(tokens_hbm.at[wid, b], idx_v.at[b])
      pltpu.async_copy(table_hbm.at[idx_v.at[b]], rows[b], sg[b])

    def body(c2, carry):
      for b in (0, 1):
        c = 2 * c2 + b
        pltpu.make_async_copy(table_hbm.at[idx_v.at[b]], rows[b], sg[b]).wait()

        @pl.when(c + 2 < N_CHUNKS)
        def _():
          pltpu.sync_copy(tokens_hbm.at[wid, c + 2], idx_v.at[b])

        @pl.when(c2 > 0)
        def _():
          pltpu.make_async_copy(dense[b].at[pl.ds(0, DENSE)], out_slice(c),
                                ss[b]).wait()

        repack(rows[b], dense[b])

        @pl.when(c + 2 < N_CHUNKS)
        def _():
          pltpu.async_copy(table_hbm.at[idx_v.at[b]], rows[b], sg[b])

        pltpu.async_copy(dense[b].at[pl.ds(0, DENSE)], out_slice(c), ss[b])
      return carry

    lax.fori_loop(0, N_CHUNKS // 2, body, 0)

    for b in (0, 1):
      pltpu.make_async_copy(dense[b].at[pl.ds(0, DENSE)],
                            out_slice(N_CHUNKS - 2 + b), ss[b]).wait()

  return gather_kernel


_sc_gather = _make_sc_gather()


def kernel(tokens, table):
  table_p = jnp.pad(table, ((0, 0), (0, DP - EMBED_DIM)))
  tokens_r = tokens.reshape(NW, N_CHUNKS, CHUNK)
  out = _sc_gather(tokens_r, table_p)
  return out.reshape(BATCH, SEQ, EMBED_DIM)


# speedup vs baseline: 1.2516x; 1.0889x over previous
import functools

import jax
import jax.numpy as jnp
from jax import lax
from jax.experimental import pallas as pl
from jax.experimental.pallas import tpu as pltpu
from jax.experimental.pallas import tpu_sc as plsc

VOCAB = 100000
EMBED_DIM = 300
DP = 304
BATCH = 4096
SEQ = 200

NUM_CORES = 2
NUM_SUBCORES = 16
NW = NUM_CORES * NUM_SUBCORES

TOTAL = BATCH * SEQ
B_PER_W = TOTAL // NW
CHUNK = 100
N_CHUNKS = B_PER_W // CHUNK
DENSE = CHUNK * EMBED_DIM

_MESH = plsc.VectorSubcoreMesh(
    core_axis_name="c", subcore_axis_name="s",
    num_cores=NUM_CORES, num_subcores=NUM_SUBCORES)


def _wid():
  return lax.axis_index("s") * NUM_CORES + lax.axis_index("c")


def _repitch_rows(src, dst, nrows, src_pitch, dst_pitch, width):
  n_full = width // 16
  tail = width - 16 * n_full

  def row_body(r, carry):
    db = r * dst_pitch
    for k in range(n_full):
      dst[pl.ds(db + 16 * k, 16)] = src[r, pl.ds(16 * k, 16)]
    if tail:
      dst[pl.ds(db + width - 16, 16)] = src[r, pl.ds(width - 16, 16)]
    return carry

  lax.fori_loop(0, nrows, row_body, 0)


_G1 = 16
_N_G1 = BATCH // _G1
_G1_PER_W = _N_G1 // NW


@functools.partial(
    pl.kernel,
    mesh=_MESH,
    out_type=jax.ShapeDtypeStruct((TOTAL,), jnp.int32),
    scratch_types=[
        pltpu.VMEM((_G1, SEQ), jnp.int32),
        pltpu.VMEM((_G1 * SEQ,), jnp.int32),
    ],
    compiler_params=pltpu.CompilerParams(use_tc_tiling_on_sc=True),
)
def _flatten_tokens(tokens_hbm, out_hbm, tin, tflat):
  w = _wid()

  def body(t, carry):
    g = w * _G1_PER_W + t
    pltpu.sync_copy(tokens_hbm.at[pl.ds(g * _G1, _G1)], tin)
    _repitch_rows(tin, tflat, _G1, SEQ, SEQ, SEQ)
    pltpu.sync_copy(tflat, out_hbm.at[pl.ds(g * _G1 * SEQ, _G1 * SEQ)])
    return carry

  lax.fori_loop(0, _G1_PER_W, body, 0)


_G2 = 40
_N_G2 = VOCAB // _G2
_G2_PER_W = -(-_N_G2 // NW)
_G2_IN = _G2 * EMBED_DIM
_G2_OUT = _G2 * DP


@functools.partial(
    pl.kernel,
    mesh=_MESH,
    out_type=jax.ShapeDtypeStruct((VOCAB * DP,), jnp.float32),
    scratch_types=[
        pltpu.VMEM((_G2, EMBED_DIM), jnp.float32),
        pltpu.VMEM((_G2, EMBED_DIM), jnp.float32),
        pltpu.VMEM((_G2_OUT,), jnp.float32),
        pltpu.VMEM((_G2_OUT,), jnp.float32),
        pltpu.SemaphoreType.DMA,
        pltpu.SemaphoreType.DMA,
        pltpu.SemaphoreType.DMA,
        pltpu.SemaphoreType.DMA,
    ],
    compiler_params=pltpu.CompilerParams(use_tc_tiling_on_sc=True),
)
def _pad_table(table_hbm, out_hbm, tin0, tin1, td0, td1, si0, si1, so0, so1):
  w = _wid()
  tin = (tin0, tin1)
  td = (td0, td1)
  si = (si0, si1)
  so = (so0, so1)

  def grp(t):
    return w + NW * t

  def in_slice(t):
    return table_hbm.at[pl.ds(grp(t) * _G2, _G2)]

  def out_slice(t):
    return out_hbm.at[pl.ds(grp(t) * _G2_OUT, _G2_OUT)]

  for b in (0, 1):
    @pl.when(grp(b) < _N_G2)
    def _():
      pltpu.async_copy(in_slice(b), tin[b], si[b])

  def body(t2, carry):
    for b in (0, 1):
      t = 2 * t2 + b

      @pl.when(grp(t) < _N_G2)
      def _():
        pltpu.make_async_copy(in_slice(t), tin[b], si[b]).wait()

        @pl.when(t2 > 0)
        def _():
          pltpu.make_async_copy(td[b], out_slice(t), so[b]).wait()

        _repitch_rows(tin[b], td[b], _G2, EMBED_DIM, DP, EMBED_DIM)

        @pl.when(grp(t + 2) < _N_G2)
        def _():
          pltpu.async_copy(in_slice(t + 2), tin[b], si[b])

        pltpu.async_copy(td[b], out_slice(t), so[b])
    return carry

  lax.fori_loop(0, _G2_PER_W // 2 + 1, body, 0)

  t_max = (_N_G2 - 1 - w) // NW
  for b in (0, 1):
    t_last = jnp.where(t_max % 2 == b, t_max, t_max - 1)
    pltpu.make_async_copy(td[b], out_slice(t_last), so[b]).wait()


@functools.partial(
    pl.kernel,
    mesh=_MESH,
    out_type=jax.ShapeDtypeStruct((TOTAL * EMBED_DIM,), jnp.float32),
    scratch_types=[
        pltpu.VMEM((2, CHUNK), jnp.int32),
        pltpu.VMEM((CHUNK, DP), jnp.float32),
        pltpu.VMEM((CHUNK, DP), jnp.float32),
        pltpu.VMEM((DENSE,), jnp.float32),
        pltpu.VMEM((DENSE,), jnp.float32),
        pltpu.SemaphoreType.DMA,
        pltpu.SemaphoreType.DMA,
        pltpu.SemaphoreType.DMA,
        pltpu.SemaphoreType.DMA,
    ],
    compiler_params=pltpu.CompilerParams(use_tc_tiling_on_sc=False),
)
def _sc_gather(tokens_hbm, table_hbm, out_hbm, idx_v, rows0, rows1,
               dense0, dense1, sg0, sg1, ss0, ss1):
  w = _wid()
  base = w * B_PER_W
  rows = (rows0, rows1)
  dense = (dense0, dense1)
  sg = (sg0, sg1)
  ss = (ss0, ss1)

  def out_slice(c):
    return out_hbm.at[pl.ds((base + c * CHUNK) * EMBED_DIM, DENSE)]

  for b in (0, 1):
    pltpu.sync_copy(tokens_hbm.at[w, b], idx_v.at[b])
    pltpu.async_copy(table_hbm.at[idx_v.at[b]], rows[b], sg[b])

  def body(c2, carry):
    for b in (0, 1):
      c = 2 * c2 + b
      pltpu.make_async_copy(table_hbm.at[idx_v.at[b]], rows[b], sg[b]).wait()

      @pl.when(c + 2 < N_CHUNKS)
      def _():
        pltpu.sync_copy(tokens_hbm.at[w, c + 2], idx_v.at[b])

      @pl.when(c2 > 0)
      def _():
        pltpu.make_async_copy(dense[b], out_slice(c), ss[b]).wait()

      _repitch_rows(rows[b], dense[b], CHUNK, DP, EMBED_DIM, EMBED_DIM)

      @pl.when(c + 2 < N_CHUNKS)
      def _():
        pltpu.async_copy(table_hbm.at[idx_v.at[b]], rows[b], sg[b])

      pltpu.async_copy(dense[b], out_slice(c), ss[b])
    return carry

  lax.fori_loop(0, N_CHUNKS // 2, body, 0)

  for b in (0, 1):
    pltpu.make_async_copy(dense[b], out_slice(N_CHUNKS - 2 + b), ss[b]).wait()


def kernel(tokens, table):
  tokens_flat = _flatten_tokens(tokens)
  table_flat = _pad_table(table)
  out = _sc_gather(tokens_flat.reshape(NW, N_CHUNKS, CHUNK),
                   table_flat.reshape(VOCAB, DP))
  return out.reshape(BATCH, SEQ, EMBED_DIM)
